# baseline (device time: 111581 ns/iter reference)
import jax
import jax.numpy as jnp
from jax import lax
from jax.experimental import pallas as pl
from jax.experimental.pallas import tpu as pltpu

N_DEV = 16
P_DIM = 4
Q_DIM = 4


def kernel(A, B):
    m, k = A.shape
    k2, n = B.shape
    assert k == k2
    qm = m // Q_DIM
    zm = qm // P_DIM
    nh = n // 2

    f32 = jnp.float32
    bf16 = jnp.bfloat16

    def body(a_ref, b_ref, out_ref, acc_ref, pbufR, pbufL,
             zbufR, zbufL,
             pa_sems, zr_sems, za_sems, pg_sems):
        me = lax.axis_index("i")
        p = me // Q_DIM
        q = lax.rem(me, Q_DIM)
        rightq = p * Q_DIM + lax.rem(q + 1, Q_DIM)
        leftq = p * Q_DIM + lax.rem(q + (Q_DIM - 1), Q_DIM)
        zright = lax.rem(p + 1, P_DIM) * Q_DIM + q
        zleft = lax.rem(p + (P_DIM - 1), P_DIM) * Q_DIM + q

        barrier_sem = pltpu.get_barrier_semaphore()
        for nbr in (leftq, rightq, zleft, zright):
            pl.semaphore_signal(
                barrier_sem, inc=1,
                device_id=(nbr,), device_id_type=pl.DeviceIdType.MESH,
            )
        pl.semaphore_wait(barrier_sem, 4)

        def qrows(c):
            return pl.ds(c * qm, qm)

        def zrows(c):
            return pl.ds(c * zm, zm)

        def mm_quarter(c):
            acc_ref[qrows(c), :] = jnp.dot(
                a_ref[qrows(c), :].astype(bf16),
                b_ref[:, :].astype(bf16),
                preferred_element_type=f32,
            ).astype(bf16)

        oRq = lax.rem(q + 1, Q_DIM)
        oLq = lax.rem(q + (Q_DIM - 1), Q_DIM)
        ozR = lax.rem(p + 1, P_DIM)
        ozL = lax.rem(p + (P_DIM - 1), P_DIM)

        def tR(j):
            if j < P_DIM - 1:
                return lax.rem(p - j + P_DIM, P_DIM)
            return lax.rem(p + 1, P_DIM)

        def tL(j):
            if j < P_DIM - 1:
                return lax.rem(p + j, P_DIM)
            return lax.rem(p + (P_DIM - 1), P_DIM)

        def mm_slab(cq, cz):
            rows = pl.ds(cq * qm + cz * zm, zm)
            acc_ref[rows, :] = jnp.dot(
                a_ref[rows, :].astype(bf16),
                b_ref[:, :].astype(bf16),
                preferred_element_type=f32,
            ).astype(bf16)

        prs = {}

        def plane_rs_hop(d, j, h, src, t):
            if d == "R":
                src_ref = (acc_ref.at[pl.ds(q * qm + t * zm, zm), :nh]
                           if src is None else pbufR.at[src, zrows(t), :])
                desc = pltpu.make_async_remote_copy(
                    src_ref=src_ref,
                    dst_ref=pbufR.at[h, zrows(t), :],
                    send_sem=pa_sems.at[j, h, 0], recv_sem=pa_sems.at[j, h, 1],
                    device_id=(rightq,), device_id_type=pl.DeviceIdType.MESH,
                )
            else:
                src_ref = (acc_ref.at[pl.ds(q * qm + t * zm, zm), nh:]
                           if src is None else pbufL.at[src, zrows(t), :])
                desc = pltpu.make_async_remote_copy(
                    src_ref=src_ref,
                    dst_ref=pbufL.at[h, zrows(t), :],
                    send_sem=pa_sems.at[j, h, 2], recv_sem=pa_sems.at[j, h, 3],
                    device_id=(leftq,), device_id_type=pl.DeviceIdType.MESH,
                )
            desc.start()
            prs[(d, j, h)] = desc

        mm_slab(q, lax.rem(p, P_DIM))
        plane_rs_hop("R", 0, 0, None, tR(0))
        plane_rs_hop("L", 0, 0, None, tL(0))
        mm_slab(q, lax.rem(p + (P_DIM - 1), P_DIM))
        plane_rs_hop("R", 1, 0, None, tR(1))
        mm_slab(q, lax.rem(p + 1, P_DIM))
        plane_rs_hop("L", 1, 0, None, tL(1))
        mm_slab(q, lax.rem(p + 2, P_DIM))
        plane_rs_hop("R", 2, 0, None, tR(2))
        plane_rs_hop("L", 2, 0, None, tL(2))
        plane_rs_hop("R", 3, 0, None, tR(3))
        plane_rs_hop("L", 3, 0, None, tL(3))
        mm_quarter(lax.rem(q + 1, Q_DIM))
        mm_quarter(lax.rem(q + (Q_DIM - 1), Q_DIM))

        def rs_step(d, j, h):
            if d == "R":
                t = tR(j)
                qh = lax.rem(q - h + Q_DIM, Q_DIM)
                prs[("R", j, h - 1)].wait_recv()
                pbufR[h - 1, zrows(t), :] = (
                    pbufR[h - 1, zrows(t), :]
                    + acc_ref[pl.ds(qh * qm + t * zm, zm), :nh]
                )
                plane_rs_hop("R", j, h, h - 1, t)
            else:
                t = tL(j)
                qh = lax.rem(q + h, Q_DIM)
                prs[("L", j, h - 1)].wait_recv()
                pbufL[h - 1, zrows(t), :] = (
                    pbufL[h - 1, zrows(t), :]
                    + acc_ref[pl.ds(qh * qm + t * zm, zm), nh:]
                )
                plane_rs_hop("L", j, h, h - 1, t)

        def plane_final_R(c):
            return (pbufR[Q_DIM - 2, zrows(c), :]
                    + acc_ref[pl.ds(oRq * qm + c * zm, zm), :nh])

        def plane_final_L(c):
            return (pbufL[Q_DIM - 2, zrows(c), :]
                    + acc_ref[pl.ds(oLq * qm + c * zm, zm), nh:])

        zrs = {}

        def z_rs_hop(d, s, src_slot):
            if d == "R":
                desc = pltpu.make_async_remote_copy(
                    src_ref=zbufR.at[src_slot], dst_ref=zbufR.at[s],
                    send_sem=zr_sems.at[s, 0], recv_sem=zr_sems.at[s, 1],
                    device_id=(zright,), device_id_type=pl.DeviceIdType.MESH,
                )
            else:
                desc = pltpu.make_async_remote_copy(
                    src_ref=zbufL.at[src_slot], dst_ref=zbufL.at[s],
                    send_sem=zr_sems.at[s, 2], recv_sem=zr_sems.at[s, 3],
                    device_id=(zleft,), device_id_type=pl.DeviceIdType.MESH,
                )
            desc.start()
            zrs[(d, s)] = desc

        def z_step(d, s):
            if s == 0:
                prs[(d, 0, 2)].wait_recv()
                if d == "R":
                    zbufR[P_DIM - 1, :, :] = plane_final_R(tR(0))
                else:
                    zbufL[P_DIM - 1, :, :] = plane_final_L(tL(0))
                z_rs_hop(d, 0, P_DIM - 1)
            else:
                zrs[(d, s - 1)].wait_recv()
                prs[(d, s, 2)].wait_recv()
                if d == "R":
                    zbufR[s - 1, :, :] = zbufR[s - 1, :, :] + plane_final_R(tR(s))
                else:
                    zbufL[s - 1, :, :] = zbufL[s - 1, :, :] + plane_final_L(tL(s))
                z_rs_hop(d, s, s - 1)

        q2 = lax.rem(q + 2, Q_DIM)
        rs_step("R", 0, 1)
        rs_step("L", 0, 1)
        rs_step("R", 1, 1)
        rs_step("L", 1, 1)
        mm_slab(q2, lax.rem(p, P_DIM))
        rs_step("R", 0, 2)
        rs_step("L", 0, 2)
        rs_step("R", 2, 1)
        rs_step("L", 2, 1)
        mm_slab(q2, lax.rem(p + (P_DIM - 1), P_DIM))
        mm_slab(q2, lax.rem(p + 1, P_DIM))
        rs_step("R", 1, 2)
        rs_step("L", 1, 2)
        rs_step("R", 3, 1)
        rs_step("L", 3, 1)
        mm_slab(q2, lax.rem(p + 2, P_DIM))
        z_step("R", 0)
        z_step("L", 0)
        rs_step("R", 2, 2)
        rs_step("L", 2, 2)
        z_step("R", 1)
        z_step("L", 1)
        rs_step("R", 3, 2)
        rs_step("L", 3, 2)
        z_step("R", 2)
        z_step("L", 2)

        def gelu(z):
            return 0.5 * z * (
                1.0 + jnp.tanh(0.7978845608 * (z + 0.044715 * z * z * z))
            )

        def z_hop(d, s):
            if d == "R":
                g = lax.rem(p + 1 - s + P_DIM, P_DIM)
                rows = pl.ds(oRq * qm + g * zm, zm)
                desc = pltpu.make_async_remote_copy(
                    src_ref=out_ref.at[rows, :nh],
                    dst_ref=out_ref.at[rows, :nh],
                    send_sem=za_sems.at[s, 0], recv_sem=za_sems.at[s, 1],
                    device_id=(zright,), device_id_type=pl.DeviceIdType.MESH,
                )
            else:
                g = lax.rem(p - 1 + s + P_DIM, P_DIM)
                rows = pl.ds(oLq * qm + g * zm, zm)
                desc = pltpu.make_async_remote_copy(
                    src_ref=out_ref.at[rows, nh:],
                    dst_ref=out_ref.at[rows, nh:],
                    send_sem=za_sems.at[s, 2], recv_sem=za_sems.at[s, 3],
                    device_id=(zleft,), device_id_type=pl.DeviceIdType.MESH,
                )
            desc.start()
            return desc

        def plane_hop(d, r, h):
            if d == "R":
                cz = lax.rem(p + 1 - r + P_DIM, P_DIM)
                qh = lax.rem(q + 1 - h + Q_DIM, Q_DIM)
                rows = pl.ds(qh * qm + cz * zm, zm)
                desc = pltpu.make_async_remote_copy(
                    src_ref=out_ref.at[rows, :nh],
                    dst_ref=out_ref.at[rows, :nh],
                    send_sem=pg_sems.at[r, h, 0], recv_sem=pg_sems.at[r, h, 1],
                    device_id=(rightq,), device_id_type=pl.DeviceIdType.MESH,
                )
            else:
                cz = lax.rem(p - 1 + r + P_DIM, P_DIM)
                qh = lax.rem(q - 1 + h + Q_DIM, Q_DIM)
                rows = pl.ds(qh * qm + cz * zm, zm)
                desc = pltpu.make_async_remote_copy(
                    src_ref=out_ref.at[rows, nh:],
                    dst_ref=out_ref.at[rows, nh:],
                    send_sem=pg_sems.at[r, h, 2], recv_sem=pg_sems.at[r, h, 3],
                    device_id=(leftq,), device_id_type=pl.DeviceIdType.MESH,
                )
            desc.start()
            return desc

        zd = {}
        pr = {}
        zrs[("R", 2)].wait_recv()
        prs[("R", 3, 2)].wait_recv()
        zR = zbufR[P_DIM - 2, :, :].astype(f32) + plane_final_R(ozR)
        out_ref[pl.ds(oRq * qm + ozR * zm, zm), :nh] = gelu(zR).astype(bf16)
        zrs[("L", 2)].wait_recv()
        prs[("L", 3, 2)].wait_recv()
        zL = zbufL[P_DIM - 2, :, :].astype(f32) + plane_final_L(ozL)
        out_ref[pl.ds(oLq * qm + ozL * zm, zm), nh:] = gelu(zL).astype(bf16)
        zd[(0, "R")] = z_hop("R", 0)
        pr[(0, 0, "R")] = plane_hop("R", 0, 0)
        zd[(0, "L")] = z_hop("L", 0)
        pr[(0, 0, "L")] = plane_hop("L", 0, 0)
        oppq = p * Q_DIM + lax.rem(q + 2, Q_DIM)

        def direct_r3(d):
            cz = lax.rem(p + 2, P_DIM)
            if d == "R":
                rows = pl.ds(oRq * qm + cz * zm, zm)
            else:
                rows = pl.ds(oLq * qm + cz * zm, zm)
            for i, tgt in enumerate((rightq, leftq, oppq)):
                if d == "R":
                    desc = pltpu.make_async_remote_copy(
                        src_ref=out_ref.at[rows, :nh],
                        dst_ref=out_ref.at[rows, :nh],
                        send_sem=pg_sems.at[3, i, 0],
                        recv_sem=pg_sems.at[3, i, 1],
                        device_id=(tgt,), device_id_type=pl.DeviceIdType.MESH,
                    )
                else:
                    desc = pltpu.make_async_remote_copy(
                        src_ref=out_ref.at[rows, nh:],
                        dst_ref=out_ref.at[rows, nh:],
                        send_sem=pg_sems.at[3, i, 2],
                        recv_sem=pg_sems.at[3, i, 3],
                        device_id=(tgt,), device_id_type=pl.DeviceIdType.MESH,
                    )
                desc.start()
                pr[(3, i, d)] = desc

        for s in (1, 2):
            for d in ("R", "L"):
                zd[(s - 1, d)].wait_recv()
                zd[(s, d)] = z_hop(d, s)
                pr[(s, 0, d)] = plane_hop(d, s, 0)
        for d in ("R", "L"):
            zd[(2, d)].wait_recv()
            direct_r3(d)
        for h in (1, 2):
            for r in range(P_DIM - 1):
                for d in ("R", "L"):
                    pr[(r, h - 1, d)].wait_recv()
                    pr[(r, h, d)] = plane_hop(d, r, h)
        for r in range(P_DIM - 1):
            pr[(r, 2, "R")].wait_recv()
            pr[(r, 2, "L")].wait_recv()
        for i in range(3):
            pr[(3, i, "R")].wait_recv()
            pr[(3, i, "L")].wait_recv()
        for desc in (list(zd.values()) + list(pr.values())
                     + list(prs.values()) + list(zrs.values())):
            desc.wait_send()

    return pl.pallas_call(
        body,
        out_shape=jax.ShapeDtypeStruct((m, n), bf16),
        in_specs=[
            pl.BlockSpec(memory_space=pltpu.VMEM),
            pl.BlockSpec(memory_space=pltpu.VMEM),
        ],
        out_specs=pl.BlockSpec(memory_space=pltpu.VMEM),
        scratch_shapes=[
            pltpu.VMEM((m, n), bf16),
            pltpu.VMEM((Q_DIM, qm, nh), bf16),
            pltpu.VMEM((Q_DIM, qm, nh), bf16),
            pltpu.VMEM((P_DIM, zm, nh), bf16),
            pltpu.VMEM((P_DIM, zm, nh), bf16),
            pltpu.SemaphoreType.DMA((P_DIM, Q_DIM - 1, 4)),
            pltpu.SemaphoreType.DMA((P_DIM - 1, 4)),
            pltpu.SemaphoreType.DMA((P_DIM - 1, 4)),
            pltpu.SemaphoreType.DMA((P_DIM, Q_DIM - 1, 4)),
        ],
        compiler_params=pltpu.CompilerParams(
            collective_id=0,
            vmem_limit_bytes=100 * 1024 * 1024,
        ),
    )(A, B)


# device time: 110446 ns/iter; 1.0103x vs baseline; 1.0103x over previous
import jax
import jax.numpy as jnp
from jax import lax
from jax.experimental import pallas as pl
from jax.experimental.pallas import tpu as pltpu

N_DEV = 16
P_DIM = 4
Q_DIM = 4


def kernel(A, B):
    m, k = A.shape
    k2, n = B.shape
    assert k == k2
    qm = m // Q_DIM
    zm = qm // P_DIM
    nh = n // 2

    f32 = jnp.float32
    bf16 = jnp.bfloat16

    def body(a_ref, b_ref, out_ref, acc_ref, pbufR, pbufL,
             zbufR, zbufL,
             pa_sems, zr_sems, za_sems, pg_sems):
        me = lax.axis_index("i")
        p = me // Q_DIM
        q = lax.rem(me, Q_DIM)
        rightq = p * Q_DIM + lax.rem(q + 1, Q_DIM)
        leftq = p * Q_DIM + lax.rem(q + (Q_DIM - 1), Q_DIM)
        zright = lax.rem(p + 1, P_DIM) * Q_DIM + q
        zleft = lax.rem(p + (P_DIM - 1), P_DIM) * Q_DIM + q

        barrier_sem = pltpu.get_barrier_semaphore()
        for nbr in (leftq, rightq, zleft, zright):
            pl.semaphore_signal(
                barrier_sem, inc=1,
                device_id=(nbr,), device_id_type=pl.DeviceIdType.MESH,
            )
        pl.semaphore_wait(barrier_sem, 4)

        def qrows(c):
            return pl.ds(c * qm, qm)

        def zrows(c):
            return pl.ds(c * zm, zm)

        def mm_quarter(c):
            acc_ref[qrows(c), :] = jnp.dot(
                a_ref[qrows(c), :].astype(bf16),
                b_ref[:, :].astype(bf16),
                preferred_element_type=f32,
            ).astype(bf16)

        oRq = lax.rem(q + 1, Q_DIM)
        oLq = lax.rem(q + (Q_DIM - 1), Q_DIM)
        ozR = lax.rem(p + 1, P_DIM)
        ozL = lax.rem(p + (P_DIM - 1), P_DIM)

        def tR(j):
            if j < P_DIM - 1:
                return lax.rem(p - j + P_DIM, P_DIM)
            return lax.rem(p + 1, P_DIM)

        def tL(j):
            if j < P_DIM - 1:
                return lax.rem(p + j, P_DIM)
            return lax.rem(p + (P_DIM - 1), P_DIM)

        def mm_slab(cq, cz):
            rows = pl.ds(cq * qm + cz * zm, zm)
            acc_ref[rows, :] = jnp.dot(
                a_ref[rows, :].astype(bf16),
                b_ref[:, :].astype(bf16),
                preferred_element_type=f32,
            ).astype(bf16)

        prs = {}

        def plane_rs_hop(d, j, h, src, t):
            if d == "R":
                src_ref = (acc_ref.at[pl.ds(q * qm + t * zm, zm), :nh]
                           if src is None else pbufR.at[src, zrows(t), :])
                desc = pltpu.make_async_remote_copy(
                    src_ref=src_ref,
                    dst_ref=pbufR.at[h, zrows(t), :],
                    send_sem=pa_sems.at[j, h, 0], recv_sem=pa_sems.at[j, h, 1],
                    device_id=(rightq,), device_id_type=pl.DeviceIdType.MESH,
                )
            else:
                src_ref = (acc_ref.at[pl.ds(q * qm + t * zm, zm), nh:]
                           if src is None else pbufL.at[src, zrows(t), :])
                desc = pltpu.make_async_remote_copy(
                    src_ref=src_ref,
                    dst_ref=pbufL.at[h, zrows(t), :],
                    send_sem=pa_sems.at[j, h, 2], recv_sem=pa_sems.at[j, h, 3],
                    device_id=(leftq,), device_id_type=pl.DeviceIdType.MESH,
                )
            desc.start()
            prs[(d, j, h)] = desc

        mm_slab(q, lax.rem(p, P_DIM))
        plane_rs_hop("R", 0, 0, None, tR(0))
        plane_rs_hop("L", 0, 0, None, tL(0))
        mm_slab(q, lax.rem(p + (P_DIM - 1), P_DIM))
        plane_rs_hop("R", 1, 0, None, tR(1))
        mm_slab(q, lax.rem(p + 1, P_DIM))
        plane_rs_hop("L", 1, 0, None, tL(1))
        mm_slab(q, lax.rem(p + 2, P_DIM))
        plane_rs_hop("R", 2, 0, None, tR(2))
        plane_rs_hop("L", 2, 0, None, tL(2))
        plane_rs_hop("R", 3, 0, None, tR(3))
        plane_rs_hop("L", 3, 0, None, tL(3))
        mm_quarter(lax.rem(q + 1, Q_DIM))
        mm_quarter(lax.rem(q + (Q_DIM - 1), Q_DIM))

        def rs_step(d, j, h):
            if d == "R":
                t = tR(j)
                qh = lax.rem(q - h + Q_DIM, Q_DIM)
                prs[("R", j, h - 1)].wait_recv()
                pbufR[h - 1, zrows(t), :] = (
                    pbufR[h - 1, zrows(t), :]
                    + acc_ref[pl.ds(qh * qm + t * zm, zm), :nh]
                )
                plane_rs_hop("R", j, h, h - 1, t)
            else:
                t = tL(j)
                qh = lax.rem(q + h, Q_DIM)
                prs[("L", j, h - 1)].wait_recv()
                pbufL[h - 1, zrows(t), :] = (
                    pbufL[h - 1, zrows(t), :]
                    + acc_ref[pl.ds(qh * qm + t * zm, zm), nh:]
                )
                plane_rs_hop("L", j, h, h - 1, t)

        def plane_final_R(c):
            return (pbufR[Q_DIM - 2, zrows(c), :]
                    + acc_ref[pl.ds(oRq * qm + c * zm, zm), :nh])

        def plane_final_L(c):
            return (pbufL[Q_DIM - 2, zrows(c), :]
                    + acc_ref[pl.ds(oLq * qm + c * zm, zm), nh:])

        zrs = {}

        def z_rs_hop(d, s, src_slot):
            if d == "R":
                desc = pltpu.make_async_remote_copy(
                    src_ref=zbufR.at[src_slot], dst_ref=zbufR.at[s],
                    send_sem=zr_sems.at[s, 0], recv_sem=zr_sems.at[s, 1],
                    device_id=(zright,), device_id_type=pl.DeviceIdType.MESH,
                )
            else:
                desc = pltpu.make_async_remote_copy(
                    src_ref=zbufL.at[src_slot], dst_ref=zbufL.at[s],
                    send_sem=zr_sems.at[s, 2], recv_sem=zr_sems.at[s, 3],
                    device_id=(zleft,), device_id_type=pl.DeviceIdType.MESH,
                )
            desc.start()
            zrs[(d, s)] = desc

        def z_step(d, s):
            if s == 0:
                prs[(d, 0, 2)].wait_recv()
                if d == "R":
                    zbufR[P_DIM - 1, :, :] = plane_final_R(tR(0))
                else:
                    zbufL[P_DIM - 1, :, :] = plane_final_L(tL(0))
                z_rs_hop(d, 0, P_DIM - 1)
            else:
                zrs[(d, s - 1)].wait_recv()
                prs[(d, s, 2)].wait_recv()
                if d == "R":
                    zbufR[s - 1, :, :] = zbufR[s - 1, :, :] + plane_final_R(tR(s))
                else:
                    zbufL[s - 1, :, :] = zbufL[s - 1, :, :] + plane_final_L(tL(s))
                z_rs_hop(d, s, s - 1)

        q2 = lax.rem(q + 2, Q_DIM)
        rs_step("R", 0, 1)
        rs_step("L", 0, 1)
        rs_step("R", 1, 1)
        rs_step("L", 1, 1)
        mm_slab(q2, lax.rem(p, P_DIM))
        rs_step("R", 0, 2)
        rs_step("L", 0, 2)
        rs_step("R", 2, 1)
        rs_step("L", 2, 1)
        mm_slab(q2, lax.rem(p + (P_DIM - 1), P_DIM))
        mm_slab(q2, lax.rem(p + 1, P_DIM))
        rs_step("R", 1, 2)
        rs_step("L", 1, 2)
        rs_step("R", 3, 1)
        rs_step("L", 3, 1)
        mm_slab(q2, lax.rem(p + 2, P_DIM))
        z_step("R", 0)
        z_step("L", 0)
        rs_step("R", 2, 2)
        rs_step("L", 2, 2)
        z_step("R", 1)
        z_step("L", 1)
        rs_step("R", 3, 2)
        rs_step("L", 3, 2)
        z_step("R", 2)
        z_step("L", 2)

        def gelu(z):
            return 0.5 * z * (
                1.0 + jnp.tanh(0.7978845608 * (z + 0.044715 * z * z * z))
            )

        def z_hop(d, s):
            if d == "R":
                g = lax.rem(p + 1 - s + P_DIM, P_DIM)
                rows = pl.ds(oRq * qm + g * zm, zm)
                desc = pltpu.make_async_remote_copy(
                    src_ref=out_ref.at[rows, :nh],
                    dst_ref=out_ref.at[rows, :nh],
                    send_sem=za_sems.at[s, 0], recv_sem=za_sems.at[s, 1],
                    device_id=(zright,), device_id_type=pl.DeviceIdType.MESH,
                )
            else:
                g = lax.rem(p - 1 + s + P_DIM, P_DIM)
                rows = pl.ds(oLq * qm + g * zm, zm)
                desc = pltpu.make_async_remote_copy(
                    src_ref=out_ref.at[rows, nh:],
                    dst_ref=out_ref.at[rows, nh:],
                    send_sem=za_sems.at[s, 2], recv_sem=za_sems.at[s, 3],
                    device_id=(zleft,), device_id_type=pl.DeviceIdType.MESH,
                )
            desc.start()
            return desc

        def plane_hop(d, r, h):
            if d == "R":
                cz = lax.rem(p + 1 - r + P_DIM, P_DIM)
                qh = lax.rem(q + 1 - h + Q_DIM, Q_DIM)
                rows = pl.ds(qh * qm + cz * zm, zm)
                desc = pltpu.make_async_remote_copy(
                    src_ref=out_ref.at[rows, :nh],
                    dst_ref=out_ref.at[rows, :nh],
                    send_sem=pg_sems.at[r, h, 0], recv_sem=pg_sems.at[r, h, 1],
                    device_id=(rightq,), device_id_type=pl.DeviceIdType.MESH,
                )
            else:
                cz = lax.rem(p - 1 + r + P_DIM, P_DIM)
                qh = lax.rem(q - 1 + h + Q_DIM, Q_DIM)
                rows = pl.ds(qh * qm + cz * zm, zm)
                desc = pltpu.make_async_remote_copy(
                    src_ref=out_ref.at[rows, nh:],
                    dst_ref=out_ref.at[rows, nh:],
                    send_sem=pg_sems.at[r, h, 2], recv_sem=pg_sems.at[r, h, 3],
                    device_id=(leftq,), device_id_type=pl.DeviceIdType.MESH,
                )
            desc.start()
            return desc

        zd = {}
        pr = {}
        zrs[("R", 2)].wait_recv()
        prs[("R", 3, 2)].wait_recv()
        zR = zbufR[P_DIM - 2, :, :].astype(f32) + plane_final_R(ozR)
        out_ref[pl.ds(oRq * qm + ozR * zm, zm), :nh] = gelu(zR).astype(bf16)
        zrs[("L", 2)].wait_recv()
        prs[("L", 3, 2)].wait_recv()
        zL = zbufL[P_DIM - 2, :, :].astype(f32) + plane_final_L(ozL)
        out_ref[pl.ds(oLq * qm + ozL * zm, zm), nh:] = gelu(zL).astype(bf16)
        zd[(0, "R")] = z_hop("R", 0)
        pr[(0, 0, "R")] = plane_hop("R", 0, 0)
        zd[(0, "L")] = z_hop("L", 0)
        pr[(0, 0, "L")] = plane_hop("L", 0, 0)
        for s in (1, 2):
            for d in ("R", "L"):
                zd[(s - 1, d)].wait_recv()
                zd[(s, d)] = z_hop(d, s)
                pr[(s, 0, d)] = plane_hop(d, s, 0)
        for d in ("R", "L"):
            zd[(2, d)].wait_recv()
            pr[(3, 0, d)] = plane_hop(d, 3, 0)
        for h in (1, 2):
            for r in range(P_DIM):
                for d in ("R", "L"):
                    pr[(r, h - 1, d)].wait_recv()
                    pr[(r, h, d)] = plane_hop(d, r, h)
        for r in range(P_DIM):
            pr[(r, 2, "R")].wait_recv()
            pr[(r, 2, "L")].wait_recv()
        for desc in (list(zd.values()) + list(pr.values())
                     + list(prs.values()) + list(zrs.values())):
            desc.wait_send()

    return pl.pallas_call(
        body,
        out_shape=jax.ShapeDtypeStruct((m, n), bf16),
        in_specs=[
            pl.BlockSpec(memory_space=pltpu.VMEM),
            pl.BlockSpec(memory_space=pltpu.VMEM),
        ],
        out_specs=pl.BlockSpec(memory_space=pltpu.VMEM),
        scratch_shapes=[
            pltpu.VMEM((m, n), bf16),
            pltpu.VMEM((Q_DIM, qm, nh), bf16),
            pltpu.VMEM((Q_DIM, qm, nh), bf16),
            pltpu.VMEM((P_DIM, zm, nh), bf16),
            pltpu.VMEM((P_DIM, zm, nh), bf16),
            pltpu.SemaphoreType.DMA((P_DIM, Q_DIM - 1, 4)),
            pltpu.SemaphoreType.DMA((P_DIM - 1, 4)),
            pltpu.SemaphoreType.DMA((P_DIM - 1, 4)),
            pltpu.SemaphoreType.DMA((P_DIM, Q_DIM - 1, 4)),
        ],
        compiler_params=pltpu.CompilerParams(
            collective_id=0,
            vmem_limit_bytes=100 * 1024 * 1024,
        ),
    )(A, B)
